# baseline (speedup 1.0000x reference)
import functools

import jax
import jax.numpy as jnp
from jax import lax
from jax.experimental import pallas as pl
from jax.experimental.pallas import tpu as pltpu
from jax.experimental.pallas import tpu_sc as plsc

_NC = 2
_NS = 16
_NW = _NC * _NS
_L = 16


def _add_block(x_ref, t_ref, o_ref):
    o_ref[...] = x_ref[...] + t_ref[...]


def _tc_add(x, pos_table):
    B, S, D = x.shape
    BS = 2048
    grid = (S // BS, B)
    return pl.pallas_call(
        _add_block,
        grid=grid,
        in_specs=[
            pl.BlockSpec((1, BS, D), lambda i, b: (b, i, 0)),
            pl.BlockSpec((BS, D), lambda i, b: (i, 0)),
        ],
        out_specs=pl.BlockSpec((1, BS, D), lambda i, b: (b, i, 0)),
        out_shape=jax.ShapeDtypeStruct((B, S, D), x.dtype),
    )(x, pos_table)


def _sc_copy(R, D, C):
    RPW = R // _NW
    NCHUNK = RPW // C
    CW = C * D
    mesh = plsc.VectorSubcoreMesh(core_axis_name="c", subcore_axis_name="s")

    @functools.partial(
        pl.kernel, mesh=mesh,
        out_type=jax.ShapeDtypeStruct((R * D,), jnp.float32),
        scratch_types=[pltpu.VMEM((CW,), jnp.float32)],
    )
    def k(x_hbm, o_hbm, xv):
        wid = lax.axis_index("s") * _NC + lax.axis_index("c")
        base = wid * RPW * D

        def body(ci, carry):
            off = base + ci * CW
            pltpu.sync_copy(x_hbm.at[pl.ds(off, CW)], xv)
            pltpu.sync_copy(xv, o_hbm.at[pl.ds(off, CW)])
            return carry

        lax.fori_loop(0, NCHUNK, body, 0)

    return k


def kernel(x, pos_table):
    B, S, D = x.shape
    out = _tc_add(x, pos_table[:S])
    R = 2048
    sc_out = _sc_copy(R, D, 64)(x[0, :R].reshape(R * D))
    out, _ = lax.optimization_barrier((out, sc_out))
    return out
